# baseline (speedup 1.0000x reference)
import functools

import jax
import jax.numpy as jnp
from jax import lax
from jax.experimental import pallas as pl
from jax.experimental.pallas import tpu as pltpu
from jax.experimental.pallas import tpu_sc as plsc

_N = 32768
_B = 128
_C = _N // 16
_KMAX = 64
_ROWS_PER = _B // 32
_INT_MIN = -2147483648
_INT_MAX = 2147483647


def _sc_run(scores_hbm, k_hbm, out_hbm, bufa, bufb, ccols_v, hist_v, k_v,
            sin_a, sin_b, sout_a, sout_b):
    info = plsc.get_sparse_core_info()
    nc = info.num_cores
    wid = lax.axis_index("s") * nc + lax.axis_index("c")

    pltpu.sync_copy(k_hbm, k_v)
    kkv = k_v[...]
    lane = lax.iota(jnp.int32, 16)
    laneoff = lane * 256 + 128
    ones = jnp.ones((16,), jnp.int32)
    z16 = jnp.zeros((16,), jnp.int32)
    ninf = jnp.full((16,), -jnp.inf, jnp.float32)

    bufs = (bufa, bufb)
    sin = (sin_a, sin_b)
    sout = (sout_a, sout_b)
    for b in bufs:
        b[pl.ds(_N, 16)] = plsc.bitcast(z16 + jnp.int32(_INT_MAX),
                                        jnp.float32)

    row0 = wid * _ROWS_PER
    incp = [None, None]
    outcp = [None, None]
    incp[0] = pltpu.async_copy(scores_hbm.at[row0], bufa.at[pl.ds(0, _N)],
                               sin[0])

    for r in range(_ROWS_PER):
        p = r % 2
        q = 1 - p
        buf = bufs[p]
        incp[p].wait()
        if r >= 1:
            outcp[q].wait()
        if r + 1 < _ROWS_PER:
            incp[q] = pltpu.async_copy(scores_hbm.at[row0 + r + 1],
                                       bufs[q].at[pl.ds(0, _N)], sin[q])

        @plsc.parallel_loop(0, 256, unroll=8)
        def _zstep(i):
            hist_v[pl.ds(i * 16, 16)] = z16

        @plsc.parallel_loop(0, _C, unroll=4)
        def _p1(i):
            b = plsc.bitcast(buf[pl.ds(i * 16, 16)], jnp.int32)
            key = jnp.where(b < 0, b ^ jnp.int32(0x7FFFFFFF), b)
            buf[pl.ds(i * 16, 16)] = plsc.bitcast(key, jnp.float32)
            idx = laneoff + lax.shift_right_arithmetic(key, 24)
            plsc.addupdate_scatter(hist_v, [idx], ones)

        def hr(g, carry):
            runv, bcnt, cbv = carry
            def hsum(l, a):
                return a + hist_v[pl.ds(l * 256 + g * 16, 16)]
            acc = lax.fori_loop(0, 16, hsum, z16)
            cum = runv + plsc.cumsum(acc)
            ltk = cum < kkv
            bcnt = bcnt + ltk.astype(jnp.int32)
            cbv = jnp.maximum(cbv, jnp.where(ltk, cum, 0))
            runv = plsc.cummax(lax.rev(cum, (0,)))
            return runv, bcnt, cbv
        _, bcnt, cbv = lax.fori_loop(0, 16, hr, (z16, z16, z16))
        bs8 = jnp.sum(bcnt) - 128
        c_below = jnp.max(cbv)
        bs8v = z16 + bs8

        @plsc.parallel_loop(0, _C, unroll=4, carry=z16)
        def runv2(i, rv):
            key = plsc.bitcast(buf[pl.ds(i * 16, 16)], jnp.int32)
            match = lax.shift_right_arithmetic(key, 24) == bs8v
            pf = plsc.cumsum(match.astype(jnp.int32))
            pos = jnp.maximum(rv + pf - 1, 0)
            plsc.store_scatter(ccols_v, [pos], lane + i * 16, mask=match)
            return rv + plsc.all_reduce_population_count(match)
        m_count = jnp.max(runv2)
        plsc.store_scatter(ccols_v, [m_count + lane], z16 + jnp.int32(_N))

        nv = lax.shift_right_logical(m_count + 15, 4)
        kbv = kkv - c_below
        def vstep(j, lo):
            mid = lo + lax.shift_left(jnp.int32(1), 23 - j)
            @plsc.parallel_loop(0, nv, unroll=2, carry=z16)
            def acc(i, a):
                cols = ccols_v[pl.ds(i * 16, 16)]
                ck = plsc.bitcast(plsc.load_gather(buf, [cols]), jnp.int32)
                return a + plsc.all_reduce_population_count(ck < mid)
            return jnp.where(acc >= kbv, lo, mid)
        lo0 = z16 + lax.shift_left(bs8, 24)
        t = lax.fori_loop(0, 24, vstep, lo0)

        @plsc.parallel_loop(0, nv, unroll=2, carry=z16)
        def c_less_in(i, a):
            cols = ccols_v[pl.ds(i * 16, 16)]
            ck = plsc.bitcast(plsc.load_gather(buf, [cols]), jnp.int32)
            return a + plsc.all_reduce_population_count(ck < t)
        mv = kbv - c_less_in
        def jstep(i, carry):
            run3, jv = carry
            cols = ccols_v[pl.ds(i * 16, 16)]
            ck = plsc.bitcast(plsc.load_gather(buf, [cols]), jnp.int32)
            eq = ck == t
            pf = run3 + plsc.cumsum(eq.astype(jnp.int32))
            jv = jnp.maximum(jv, jnp.where(eq & (pf <= mv), cols, -1))
            return run3 + plsc.all_reduce_population_count(eq), jv
        _, jv = lax.fori_loop(0, nv, jstep, (z16, z16 - 1))
        jmax = plsc.cummax(lax.rev(plsc.cummax(jv), (0,)))

        tv = jnp.where(kkv > 0, t, z16 + jnp.int32(_INT_MIN))
        jvv = jnp.where(kkv > 0, jmax, z16 - 1)

        @plsc.parallel_loop(0, _C, unroll=4)
        def _p3(i):
            key = plsc.bitcast(buf[pl.ds(i * 16, 16)], jnp.int32)
            col = lane + i * 16
            keep = (key < tv) | ((key == tv) & (col <= jvv))
            xb = jnp.where(key < 0, key ^ jnp.int32(0x7FFFFFFF), key)
            x = plsc.bitcast(xb, jnp.float32)
            buf[pl.ds(i * 16, 16)] = jnp.where(keep, x, ninf)

        outcp[p] = pltpu.async_copy(buf.at[pl.ds(0, _N)],
                                    out_hbm.at[row0 + r], sout[p])

    outcp[(_ROWS_PER - 1) % 2].wait()


def kernel(scores, k):
    kk = jnp.broadcast_to(
        jnp.clip(jnp.asarray(k, jnp.int32), 0, _KMAX), (16,))
    mesh = plsc.VectorSubcoreMesh(core_axis_name="c", subcore_axis_name="s")
    run = functools.partial(
        pl.kernel,
        mesh=mesh,
        compiler_params=pltpu.CompilerParams(needs_layout_passes=False),
        out_type=jax.ShapeDtypeStruct((_B, _N), jnp.float32),
        scratch_types=[
            pltpu.VMEM((_N + 16,), jnp.float32),
            pltpu.VMEM((_N + 16,), jnp.float32),
            pltpu.VMEM((_N + 16,), jnp.int32),
            pltpu.VMEM((16 * 256,), jnp.int32),
            pltpu.VMEM((16,), jnp.int32),
            pltpu.SemaphoreType.DMA,
            pltpu.SemaphoreType.DMA,
            pltpu.SemaphoreType.DMA,
            pltpu.SemaphoreType.DMA,
        ],
    )(_sc_run)
    return run(scores, kk)


# speedup vs baseline: 1.0246x; 1.0246x over previous
import functools

import jax
import jax.numpy as jnp
from jax import lax
from jax.experimental import pallas as pl
from jax.experimental.pallas import tpu as pltpu
from jax.experimental.pallas import tpu_sc as plsc

_N = 32768
_B = 128
_C = _N // 16
_KMAX = 64
_ROWS_PER = _B // 32
_INT_MIN = -2147483648
_INT_MAX = 2147483647


def _sc_run(scores_hbm, k_hbm, out_hbm, row_v, keys_v, ccols_v, hist_v, k_v):
    info = plsc.get_sparse_core_info()
    nc = info.num_cores
    wid = lax.axis_index("s") * nc + lax.axis_index("c")

    pltpu.sync_copy(k_hbm, k_v)
    kkv = k_v[...]
    lane = lax.iota(jnp.int32, 16)
    laneoff = lane * 256 + 128
    ones = jnp.ones((16,), jnp.int32)
    z16 = jnp.zeros((16,), jnp.int32)

    def do_row(r, _):
        row = wid * _ROWS_PER + r
        pltpu.sync_copy(scores_hbm.at[row], row_v)

        @plsc.parallel_loop(0, 256, unroll=8)
        def _zstep(i):
            hist_v[pl.ds(i * 16, 16)] = z16
        keys_v[pl.ds(_N, 16)] = z16 + jnp.int32(_INT_MAX)

        @plsc.parallel_loop(0, _C, unroll=4)
        def _p1(i):
            v = row_v[pl.ds(i * 16, 16)]
            b = plsc.bitcast(v, jnp.int32)
            key = jnp.where(b < 0, b ^ jnp.int32(0x7FFFFFFF), b)
            keys_v[pl.ds(i * 16, 16)] = key
            idx = laneoff + lax.shift_right_arithmetic(key, 24)
            plsc.addupdate_scatter(hist_v, [idx], ones)

        def hr(g, carry):
            runv, bcnt, cbv = carry
            def hsum(l, a):
                return a + hist_v[pl.ds(l * 256 + g * 16, 16)]
            acc = lax.fori_loop(0, 16, hsum, z16)
            cum = runv + plsc.cumsum(acc)
            ltk = cum < kkv
            bcnt = bcnt + ltk.astype(jnp.int32)
            cbv = jnp.maximum(cbv, jnp.where(ltk, cum, 0))
            runv = plsc.cummax(lax.rev(cum, (0,)))
            return runv, bcnt, cbv
        _, bcnt, cbv = lax.fori_loop(0, 16, hr, (z16, z16, z16))
        bs8 = jnp.sum(bcnt) - 128
        c_below = jnp.max(cbv)
        bs8v = z16 + bs8

        @plsc.parallel_loop(0, _C, unroll=4, carry=z16)
        def runv2(i, rv):
            key = keys_v[pl.ds(i * 16, 16)]
            match = lax.shift_right_arithmetic(key, 24) == bs8v
            pf = plsc.cumsum(match.astype(jnp.int32))
            pos = jnp.maximum(rv + pf - 1, 0)
            plsc.store_scatter(ccols_v, [pos], lane + i * 16, mask=match)
            return rv + plsc.all_reduce_population_count(match)
        m_count = jnp.max(runv2)
        plsc.store_scatter(ccols_v, [m_count + lane], z16 + jnp.int32(_N))

        nv = lax.shift_right_logical(m_count + 15, 4)
        kbv = kkv - c_below
        def vstep(j, lo):
            mid = lo + lax.shift_left(jnp.int32(1), 23 - j)
            @plsc.parallel_loop(0, nv, unroll=2, carry=z16)
            def acc(i, a):
                cols = ccols_v[pl.ds(i * 16, 16)]
                ck = plsc.load_gather(keys_v, [cols])
                return a + plsc.all_reduce_population_count(ck < mid)
            return jnp.where(acc >= kbv, lo, mid)
        lo0 = z16 + lax.shift_left(bs8, 24)
        t = lax.fori_loop(0, 24, vstep, lo0)

        @plsc.parallel_loop(0, nv, unroll=2, carry=z16)
        def c_less_in(i, a):
            cols = ccols_v[pl.ds(i * 16, 16)]
            ck = plsc.load_gather(keys_v, [cols])
            return a + plsc.all_reduce_population_count(ck < t)
        mv = kbv - c_less_in
        def jstep(i, carry):
            run3, jv = carry
            cols = ccols_v[pl.ds(i * 16, 16)]
            ck = plsc.load_gather(keys_v, [cols])
            eq = ck == t
            pf = run3 + plsc.cumsum(eq.astype(jnp.int32))
            jv = jnp.maximum(jv, jnp.where(eq & (pf <= mv), cols, -1))
            return run3 + plsc.all_reduce_population_count(eq), jv
        _, jv = lax.fori_loop(0, nv, jstep, (z16, z16 - 1))
        jmax = plsc.cummax(lax.rev(plsc.cummax(jv), (0,)))

        tv = jnp.where(kkv > 0, t, z16 + jnp.int32(_INT_MIN))
        jvv = jnp.where(kkv > 0, jmax, z16 - 1)

        @plsc.parallel_loop(0, _C, unroll=4)
        def _p3(i):
            key = keys_v[pl.ds(i * 16, 16)]
            col = lane + i * 16
            keep = (key < tv) | ((key == tv) & (col <= jvv))
            x = row_v[pl.ds(i * 16, 16)]
            row_v[pl.ds(i * 16, 16)] = jnp.where(keep, x,
                                                 jnp.float32(-jnp.inf))

        pltpu.sync_copy(row_v, out_hbm.at[row])
        return 0

    lax.fori_loop(0, _ROWS_PER, do_row, 0)


def kernel(scores, k):
    kk = jnp.broadcast_to(
        jnp.clip(jnp.asarray(k, jnp.int32), 0, _KMAX), (16,))
    mesh = plsc.VectorSubcoreMesh(core_axis_name="c", subcore_axis_name="s")
    run = functools.partial(
        pl.kernel,
        mesh=mesh,
        compiler_params=pltpu.CompilerParams(needs_layout_passes=False),
        out_type=jax.ShapeDtypeStruct((_B, _N), jnp.float32),
        scratch_types=[
            pltpu.VMEM((_N,), jnp.float32),
            pltpu.VMEM((_N + 16,), jnp.int32),
            pltpu.VMEM((_N + 16,), jnp.int32),
            pltpu.VMEM((16 * 256,), jnp.int32),
            pltpu.VMEM((16,), jnp.int32),
        ],
    )(_sc_run)
    return run(scores, kk)
